# baseline (speedup 1.0000x reference)
import functools

import jax
import jax.numpy as jnp
from jax import lax
from jax.experimental import pallas as pl
from jax.experimental.pallas import tpu as pltpu
from jax.experimental.pallas import tpu_sc as plsc

_D = 128
_ACC_ROWS = 10000
_ACC_ALLOC = _ACC_ROWS + 8
_SC_PARAMS = pltpu.CompilerParams(needs_layout_passes=False)


@functools.lru_cache(maxsize=None)
def _make_gather(n_rows, n_edges):
    del n_rows
    ch = 200
    nch = n_edges // ch
    assert n_edges % ch == 0
    nw = 32
    nit = (nch + nw - 1) // nw
    npair = (nit + 1) // 2
    assert nit >= 3
    mesh = plsc.VectorSubcoreMesh(core_axis_name="c", subcore_axis_name="s")

    def body(table, src, dst, gs, gd,
             ixs0, ixd0, ixs1, ixd1, rs0, rd0, rs1, rd1,
             isem0, isem1, gsem0, gsem1, wsem0, wsem1):
        cid = lax.axis_index("c")
        sid = lax.axis_index("s")
        wid = sid * 2 + cid
        sets = ((ixs0, ixd0, rs0, rd0, isem0, gsem0, wsem0),
                (ixs1, ixd1, rs1, rd1, isem1, gsem1, wsem1))

        def chunk_of(i):
            return wid + i * nw

        def valid(i):
            return chunk_of(i) < nch

        def issue_idx(i, st, pred):
            ixs, ixd, _, _, isem, _, _ = st

            @pl.when(pred)
            def _():
                base = chunk_of(i) * ch
                pltpu.async_copy(src.at[pl.ds(base, ch)], ixs, isem)
                pltpu.async_copy(dst.at[pl.ds(base, ch)], ixd, isem)

        def wait_idx(i, st, pred):
            ixs, ixd, _, _, isem, _, _ = st

            @pl.when(pred)
            def _():
                pltpu.make_async_copy(src.at[pl.ds(0, ch)], ixs, isem).wait()
                pltpu.make_async_copy(src.at[pl.ds(0, ch)], ixd, isem).wait()

        def issue_gather(i, st, pred):
            ixs, ixd, rs, rd, _, gsem, _ = st

            @pl.when(pred)
            def _():
                pltpu.async_copy(table.at[ixs], rs, gsem)
                pltpu.async_copy(table.at[ixd], rd, gsem)

        def finish_gather_write(i, st, pred):
            _, _, rs, rd, _, gsem, wsem = st

            @pl.when(pred)
            def _():
                pltpu.make_async_copy(gs.at[pl.ds(0, ch)], rs, gsem).wait()
                pltpu.make_async_copy(gs.at[pl.ds(0, ch)], rd, gsem).wait()
                base = chunk_of(i) * ch
                pltpu.async_copy(rs, gs.at[pl.ds(base, ch)], wsem)
                pltpu.async_copy(rd, gd.at[pl.ds(base, ch)], wsem)

        def wait_write(st, pred):
            _, _, rs, rd, _, _, wsem = st

            @pl.when(pred)
            def _():
                pltpu.make_async_copy(rs, gs.at[pl.ds(0, ch)], wsem).wait()
                pltpu.make_async_copy(rd, gd.at[pl.ds(0, ch)], wsem).wait()

        issue_idx(0, sets[0], valid(0))

        def pair(p, carry):
            for u in (0, 1):
                i = 2 * p + u
                st, pv = sets[u], sets[1 - u]
                wait_write(st, (i >= 2) & valid(i))
                wait_idx(i, st, valid(i))
                issue_gather(i, st, valid(i))
                finish_gather_write(i - 1, pv, (i >= 1) & valid(i - 1))
                issue_idx(i + 1, pv, valid(i + 1))
            return carry

        lax.fori_loop(0, npair, pair, 0)

        last = 2 * npair - 1
        finish_gather_write(last, sets[last % 2], valid(last))
        wait_write(sets[last % 2], valid(last))
        wait_write(sets[(last - 1) % 2], valid(last - 1))
        wait_write(sets[last % 2], valid(last - 2) & ~valid(last))

    return pl.kernel(
        body,
        out_type=(
            jax.ShapeDtypeStruct((n_edges, _D), jnp.float32),
            jax.ShapeDtypeStruct((n_edges, _D), jnp.float32),
        ),
        mesh=mesh,
        compiler_params=_SC_PARAMS,
        scratch_types=[
            pltpu.VMEM((ch,), jnp.int32),
            pltpu.VMEM((ch,), jnp.int32),
            pltpu.VMEM((ch,), jnp.int32),
            pltpu.VMEM((ch,), jnp.int32),
            pltpu.VMEM((ch, _D), jnp.float32),
            pltpu.VMEM((ch, _D), jnp.float32),
            pltpu.VMEM((ch, _D), jnp.float32),
            pltpu.VMEM((ch, _D), jnp.float32),
            pltpu.SemaphoreType.DMA,
            pltpu.SemaphoreType.DMA,
            pltpu.SemaphoreType.DMA,
            pltpu.SemaphoreType.DMA,
            pltpu.SemaphoreType.DMA,
            pltpu.SemaphoreType.DMA,
        ],
    )


@functools.lru_cache(maxsize=None)
def _make_scatter(n_edges, n_nodes):
    ch = 1600
    unroll = 10
    nv = ch // 16
    ng = nv // unroll
    assert nv % unroll == 0
    nch = n_edges // ch
    assert n_edges % ch == 0
    nit = (nch + 15) // 16
    npair = (nit + 1) // 2
    flush_at = 176
    trash = 336
    cap = 352
    mesh = plsc.VectorSubcoreMesh(core_axis_name="c", subcore_axis_name="s")

    if n_nodes % (2 * _ACC_ROWS) == 0:
        npass = n_nodes // (2 * _ACC_ROWS)
        split = None
    else:
        assert n_nodes <= 2 * _ACC_ROWS and n_nodes % 16 == 0
        npass = 1
        rows0 = min(((n_nodes + 31) // 32) * 16, _ACC_ROWS)
        split = (rows0, n_nodes - rows0)
        assert split[1] % 16 == 0

    def body(vals, dsts, zeros, agg, idxb0, idxb1, eidb, locb, rowsv, acc,
             sem, isem0, isem1):
        cid = lax.axis_index("c")
        sid = lax.axis_index("s")
        iota = lax.iota(jnp.int32, 16)
        pad_loc = _ACC_ROWS + (iota & 7)

        def init_stage():
            def w(v, carry):
                eidb[pl.ds(v * 16, 16)] = iota
                locb[pl.ds(v * 16, 16)] = pad_loc
                return carry
            lax.fori_loop(0, cap // 16, w, 0)

        def flush():
            pltpu.async_copy(vals.at[eidb], rowsv, sem).wait()
            pltpu.sync_copy(rowsv, acc.at[locb], add=True)
            init_stage()

        def scan_pass(base_row, rows_c):
            @pl.when(sid == 0)
            def _():
                pltpu.sync_copy(zeros, acc)
            plsc.subcore_barrier()
            init_stage()

            def issue_chunk(ci, buf, isem):
                k = sid + ci * 16

                @pl.when(k < nch)
                def _():
                    pltpu.async_copy(dsts.at[pl.ds(k * ch, ch)], buf, isem)

            def wait_chunk(ci, buf, isem):
                k = sid + ci * 16

                @pl.when(k < nch)
                def _():
                    pltpu.make_async_copy(
                        dsts.at[pl.ds(0, ch)], buf, isem).wait()

            def scan_chunk(ci, buf, off):
                k = sid + ci * 16
                valid = k < nch
                kc = jnp.where(valid, k, nch - 1)
                hi = jnp.where(valid, rows_c, 0)

                def group(g, off):
                    lanes = []
                    for u in range(unroll):
                        v = g * unroll + u
                        d = buf[pl.ds(v * 16, 16)]
                        lv = d - base_row
                        m = (lv >= 0) & (lv < hi)
                        mi = m.astype(jnp.int32)
                        lanes.append((v, lv, m, mi, plsc.cumsum(mi)))
                    o = off
                    for v, lv, m, mi, incl in lanes:
                        tgt = jnp.where(m, o + (incl - mi), trash + iota)
                        plsc.store_scatter(
                            eidb, [tgt], kc * ch + v * 16 + iota)
                        plsc.store_scatter(
                            locb, [tgt], jnp.where(m, lv, pad_loc))
                        o = o + incl[15]
                    fl = o >= flush_at

                    @pl.when(fl)
                    def _():
                        flush()

                    return jnp.where(fl, 0, o)

                return lax.fori_loop(0, ng, group, off)

            issue_chunk(0, idxb0, isem0)

            def chunk_pair(p, off):
                for u, (ba, sa, bb, sb) in (
                        (0, (idxb0, isem0, idxb1, isem1)),
                        (1, (idxb1, isem1, idxb0, isem0))):
                    ci = 2 * p + u
                    wait_chunk(ci, ba, sa)
                    issue_chunk(ci + 1, bb, sb)
                    off = scan_chunk(ci, ba, off)
                return off

            off = lax.fori_loop(0, npair, chunk_pair, jnp.int32(0))

            @pl.when(off > 0)
            def _():
                flush()

            plsc.subcore_barrier()

        def write_out(base_row, rows):
            g = rows // 16
            per, extra = g // 16, g % 16
            size_a, size_b = (per + 1) * 16, per * 16
            if extra:
                @pl.when(sid < extra)
                def _():
                    o = sid * size_a
                    pltpu.sync_copy(acc.at[pl.ds(o, size_a)],
                                    agg.at[pl.ds(base_row + o, size_a)])
            if per:
                @pl.when(sid >= extra)
                def _():
                    o = extra * size_a + (sid - extra) * size_b
                    pltpu.sync_copy(acc.at[pl.ds(o, size_b)],
                                    agg.at[pl.ds(base_row + o, size_b)])

        if split is None:
            def pass_body(p, carry):
                base_row = p * (2 * _ACC_ROWS) + cid * _ACC_ROWS
                scan_pass(base_row, _ACC_ROWS)
                write_out(base_row, _ACC_ROWS)
                plsc.subcore_barrier()
                return carry
            lax.fori_loop(0, npass, pass_body, 0)
        else:
            rows0, rows1 = split
            base_row = jnp.where(cid == 0, 0, rows0)
            rows_c = jnp.where(cid == 0, rows0, rows1)
            scan_pass(base_row, rows_c)

            @pl.when(cid == 0)
            def _():
                write_out(0, rows0)

            @pl.when(cid == 1)
            def _():
                write_out(rows0, rows1)

            plsc.subcore_barrier()

    return pl.kernel(
        body,
        out_type=jax.ShapeDtypeStruct((n_nodes, _D), jnp.float32),
        mesh=mesh,
        compiler_params=_SC_PARAMS,
        scratch_types=[
            pltpu.VMEM((ch,), jnp.int32),
            pltpu.VMEM((ch,), jnp.int32),
            pltpu.VMEM((cap,), jnp.int32),
            pltpu.VMEM((cap,), jnp.int32),
            pltpu.VMEM((cap, _D), jnp.float32),
            pltpu.VMEM_SHARED((_ACC_ALLOC, _D), jnp.float32),
            pltpu.SemaphoreType.DMA,
            pltpu.SemaphoreType.DMA,
            pltpu.SemaphoreType.DMA,
        ],
    )


def _bdot(a, b):
    return jnp.dot(a.astype(jnp.bfloat16), b.astype(jnp.bfloat16),
                   preferred_element_type=jnp.float32)


def _mlp_ln(parts, ws, b1, w2, b2, g, bt, res):
    h = b1
    for xp, w in zip(parts, ws):
        h = h + _bdot(xp, w)
    h = jnp.maximum(h, 0.0)
    h = _bdot(h, w2) + b2
    mu = jnp.mean(h, axis=-1, keepdims=True)
    var = jnp.mean((h - mu) ** 2, axis=-1, keepdims=True)
    return res + (h - mu) * lax.rsqrt(var + 1e-5) * g + bt


def _edge_mlp(gs, gd, e, wa, wb, wc, b1, w2, b2, g, bt):
    n = gs.shape[0]
    blk = 2000
    assert n % blk == 0
    rows = pl.BlockSpec((blk, _D), lambda i: (i, 0))
    wsp = pl.BlockSpec((_D, _D), lambda i: (0, 0))
    vsp = pl.BlockSpec((1, _D), lambda i: (0, 0))

    def bodyfn(gs_r, gd_r, e_r, wa_r, wb_r, wc_r, b1_r, w2_r, b2_r, g_r, bt_r,
               o_r):
        o_r[...] = _mlp_ln(
            (gs_r[...], gd_r[...], e_r[...]),
            (wa_r[...], wb_r[...], wc_r[...]),
            b1_r[...], w2_r[...], b2_r[...], g_r[...], bt_r[...], e_r[...])

    return pl.pallas_call(
        bodyfn,
        grid=(n // blk,),
        in_specs=[rows, rows, rows, wsp, wsp, wsp, vsp, wsp, vsp, vsp, vsp],
        out_specs=rows,
        out_shape=jax.ShapeDtypeStruct((n, _D), jnp.float32),
    )(gs, gd, e, wa, wb, wc, b1, w2, b2, g, bt)


def _node_mlp(x, agg, wa, wb, b1, w2, b2, g, bt):
    n = x.shape[0]
    blk = 2000
    assert n % blk == 0
    rows = pl.BlockSpec((blk, _D), lambda i: (i, 0))
    wsp = pl.BlockSpec((_D, _D), lambda i: (0, 0))
    vsp = pl.BlockSpec((1, _D), lambda i: (0, 0))

    def bodyfn(x_r, a_r, wa_r, wb_r, b1_r, w2_r, b2_r, g_r, bt_r, o_r):
        o_r[...] = _mlp_ln(
            (x_r[...], a_r[...].astype(jnp.float32)),
            (wa_r[...], wb_r[...]),
            b1_r[...], w2_r[...], b2_r[...], g_r[...], bt_r[...], x_r[...])

    return pl.pallas_call(
        bodyfn,
        grid=(n // blk,),
        in_specs=[rows, rows, wsp, wsp, vsp, wsp, vsp, vsp, vsp],
        out_specs=rows,
        out_shape=jax.ShapeDtypeStruct((n, _D), jnp.float32),
    )(x, agg, wa, wb, b1, w2, b2, g, bt)


def kernel(h_atm, h_bnd, h_ang, edge_index_bnd, edge_index_ang, ba, ab):
    zeros = jnp.zeros((_ACC_ALLOC, _D), jnp.float32)

    def conv(x, ei, e, p, i, pregather=None):
        src, dst = ei[0], ei[1]
        n_nodes, n_edges = x.shape[0], e.shape[0]
        if pregather is None:
            gs, gd = _make_gather(n_nodes, n_edges)(x, src, dst)
        else:
            gs, gd = pregather
        e2 = _edge_mlp(
            gs, gd, e,
            p['eW1'][i, :_D], p['eW1'][i, _D:2 * _D], p['eW1'][i, 2 * _D:],
            p['eb1'][i][None], p['eW2'][i], p['eb2'][i][None],
            p['eg'][i][None], p['ebt'][i][None])
        agg = _make_scatter(n_edges, n_nodes)(e2, dst, zeros)
        x2 = _node_mlp(
            x, agg,
            p['nW1'][i, :_D], p['nW1'][i, _D:],
            p['nb1'][i][None], p['nW2'][i], p['nb2'][i][None],
            p['ng'][i][None], p['nbt'][i][None])
        return x2, e2

    for i in range(2):
        pre = _make_gather(h_atm.shape[0], h_bnd.shape[0])(
            h_atm, edge_index_bnd[0], edge_index_bnd[1])
        h_bnd, h_ang = conv(h_bnd, edge_index_ang, h_ang, ba, i)
        h_atm, h_bnd = conv(h_atm, edge_index_bnd, h_bnd, ab, i,
                            pregather=pre)
    return (h_atm, h_bnd, h_ang)


# speedup vs baseline: 1.2147x; 1.2147x over previous
import functools

import jax
import jax.numpy as jnp
from jax import lax
from jax.experimental import pallas as pl
from jax.experimental.pallas import tpu as pltpu
from jax.experimental.pallas import tpu_sc as plsc

_D = 128
_ACC_ROWS = 10000
_ACC_ALLOC = _ACC_ROWS + 8
_SC_PARAMS = pltpu.CompilerParams(needs_layout_passes=False)


@functools.lru_cache(maxsize=None)
def _make_gather(n_rows, n_edges):
    del n_rows
    ch = 200
    nch = n_edges // ch
    assert n_edges % ch == 0
    nw = 32
    nit = (nch + nw - 1) // nw
    npair = (nit + 1) // 2
    assert nit >= 3
    mesh = plsc.VectorSubcoreMesh(core_axis_name="c", subcore_axis_name="s")

    def body(table, src, dst, gs, gd,
             ixs0, ixd0, ixs1, ixd1, rs0, rd0, rs1, rd1,
             isem0, isem1, gsem0, gsem1, wsem0, wsem1):
        cid = lax.axis_index("c")
        sid = lax.axis_index("s")
        wid = sid * 2 + cid
        sets = ((ixs0, ixd0, rs0, rd0, isem0, gsem0, wsem0),
                (ixs1, ixd1, rs1, rd1, isem1, gsem1, wsem1))

        def chunk_of(i):
            return wid + i * nw

        def valid(i):
            return chunk_of(i) < nch

        def issue_idx(i, st, pred):
            ixs, ixd, _, _, isem, _, _ = st

            @pl.when(pred)
            def _():
                base = chunk_of(i) * ch
                pltpu.async_copy(src.at[pl.ds(base, ch)], ixs, isem)
                pltpu.async_copy(dst.at[pl.ds(base, ch)], ixd, isem)

        def wait_idx(i, st, pred):
            ixs, ixd, _, _, isem, _, _ = st

            @pl.when(pred)
            def _():
                pltpu.make_async_copy(src.at[pl.ds(0, ch)], ixs, isem).wait()
                pltpu.make_async_copy(src.at[pl.ds(0, ch)], ixd, isem).wait()

        def issue_gather(i, st, pred):
            ixs, ixd, rs, rd, _, gsem, _ = st

            @pl.when(pred)
            def _():
                pltpu.async_copy(table.at[ixs], rs, gsem)
                pltpu.async_copy(table.at[ixd], rd, gsem)

        def finish_gather_write(i, st, pred):
            _, _, rs, rd, _, gsem, wsem = st

            @pl.when(pred)
            def _():
                pltpu.make_async_copy(gs.at[pl.ds(0, ch)], rs, gsem).wait()
                pltpu.make_async_copy(gs.at[pl.ds(0, ch)], rd, gsem).wait()
                base = chunk_of(i) * ch
                pltpu.async_copy(rs, gs.at[pl.ds(base, ch)], wsem)
                pltpu.async_copy(rd, gd.at[pl.ds(base, ch)], wsem)

        def wait_write(st, pred):
            _, _, rs, rd, _, _, wsem = st

            @pl.when(pred)
            def _():
                pltpu.make_async_copy(rs, gs.at[pl.ds(0, ch)], wsem).wait()
                pltpu.make_async_copy(rd, gd.at[pl.ds(0, ch)], wsem).wait()

        issue_idx(0, sets[0], valid(0))

        def pair(p, carry):
            for u in (0, 1):
                i = 2 * p + u
                st, pv = sets[u], sets[1 - u]
                wait_write(st, (i >= 2) & valid(i))
                wait_idx(i, st, valid(i))
                issue_gather(i, st, valid(i))
                finish_gather_write(i - 1, pv, (i >= 1) & valid(i - 1))
                issue_idx(i + 1, pv, valid(i + 1))
            return carry

        lax.fori_loop(0, npair, pair, 0)

        last = 2 * npair - 1
        finish_gather_write(last, sets[last % 2], valid(last))
        wait_write(sets[last % 2], valid(last))
        wait_write(sets[(last - 1) % 2], valid(last - 1))
        wait_write(sets[last % 2], valid(last - 2) & ~valid(last))

    return pl.kernel(
        body,
        out_type=(
            jax.ShapeDtypeStruct((n_edges, _D), jnp.float32),
            jax.ShapeDtypeStruct((n_edges, _D), jnp.float32),
        ),
        mesh=mesh,
        compiler_params=_SC_PARAMS,
        scratch_types=[
            pltpu.VMEM((ch,), jnp.int32),
            pltpu.VMEM((ch,), jnp.int32),
            pltpu.VMEM((ch,), jnp.int32),
            pltpu.VMEM((ch,), jnp.int32),
            pltpu.VMEM((ch, _D), jnp.float32),
            pltpu.VMEM((ch, _D), jnp.float32),
            pltpu.VMEM((ch, _D), jnp.float32),
            pltpu.VMEM((ch, _D), jnp.float32),
            pltpu.SemaphoreType.DMA,
            pltpu.SemaphoreType.DMA,
            pltpu.SemaphoreType.DMA,
            pltpu.SemaphoreType.DMA,
            pltpu.SemaphoreType.DMA,
            pltpu.SemaphoreType.DMA,
        ],
    )


@functools.lru_cache(maxsize=None)
def _make_scatter(n_edges, n_nodes):
    ch = 2000
    unroll = 5
    nv = ch // 16
    ng = nv // unroll
    assert nv % unroll == 0
    nch = n_edges // ch
    assert n_edges % ch == 0
    nit = (nch + 15) // 16
    npair = (nit + 1) // 2
    flush_at = 224
    trash = 304
    cap = 320
    mesh = plsc.VectorSubcoreMesh(core_axis_name="c", subcore_axis_name="s")

    if n_nodes % (2 * _ACC_ROWS) == 0:
        npass = n_nodes // (2 * _ACC_ROWS)
        split = None
    else:
        assert n_nodes <= 2 * _ACC_ROWS and n_nodes % 16 == 0
        npass = 1
        rows0 = min(((n_nodes + 31) // 32) * 16, _ACC_ROWS)
        split = (rows0, n_nodes - rows0)
        assert split[1] % 16 == 0

    def body(vals, dsts, zeros, agg, idxb0, idxb1, eidb, locb, rowsv, acc,
             sem, isem0, isem1):
        cid = lax.axis_index("c")
        sid = lax.axis_index("s")
        iota = lax.iota(jnp.int32, 16)
        pad_loc = _ACC_ROWS + (iota & 7)

        def init_stage():
            def w(v, carry):
                eidb[pl.ds(v * 16, 16)] = iota
                locb[pl.ds(v * 16, 16)] = pad_loc
                return carry
            lax.fori_loop(0, cap // 16, w, 0)

        def flush():
            pltpu.async_copy(vals.at[eidb], rowsv, sem).wait()
            pltpu.sync_copy(rowsv, acc.at[locb], add=True)
            init_stage()

        def scan_pass(base_row, rows_c):
            @pl.when(sid == 0)
            def _():
                pltpu.sync_copy(zeros, acc)
            plsc.subcore_barrier()
            init_stage()

            def issue_chunk(ci, buf, isem):
                k = sid + ci * 16

                @pl.when(k < nch)
                def _():
                    pltpu.async_copy(dsts.at[pl.ds(k * ch, ch)], buf, isem)

            def wait_chunk(ci, buf, isem):
                k = sid + ci * 16

                @pl.when(k < nch)
                def _():
                    pltpu.make_async_copy(
                        dsts.at[pl.ds(0, ch)], buf, isem).wait()

            def scan_chunk(ci, buf, off):
                k = sid + ci * 16
                valid = k < nch
                kc = jnp.where(valid, k, nch - 1)
                hi = jnp.where(valid, rows_c, 0)

                def group(g, off):
                    lanes = []
                    for u in range(unroll):
                        v = g * unroll + u
                        d = buf[pl.ds(v * 16, 16)]
                        lv = d - base_row
                        m = (lv >= 0) & (lv < hi)
                        mi = m.astype(jnp.int32)
                        lanes.append((v, lv, m, mi, plsc.cumsum(mi)))
                    o = off
                    for v, lv, m, mi, incl in lanes:
                        tgt = jnp.where(m, o + (incl - mi), trash + iota)
                        plsc.store_scatter(
                            eidb, [tgt], kc * ch + v * 16 + iota)
                        plsc.store_scatter(
                            locb, [tgt], jnp.where(m, lv, pad_loc))
                        o = o + incl[15]
                    fl = o >= flush_at

                    @pl.when(fl)
                    def _():
                        flush()

                    return jnp.where(fl, 0, o)

                return lax.fori_loop(0, ng, group, off)

            issue_chunk(0, idxb0, isem0)

            def chunk_pair(p, off):
                for u, (ba, sa, bb, sb) in (
                        (0, (idxb0, isem0, idxb1, isem1)),
                        (1, (idxb1, isem1, idxb0, isem0))):
                    ci = 2 * p + u
                    wait_chunk(ci, ba, sa)
                    issue_chunk(ci + 1, bb, sb)
                    off = scan_chunk(ci, ba, off)
                return off

            off = lax.fori_loop(0, npair, chunk_pair, jnp.int32(0))

            @pl.when(off > 0)
            def _():
                flush()

            plsc.subcore_barrier()

        def write_out(base_row, rows):
            g = rows // 16
            per, extra = g // 16, g % 16
            size_a, size_b = (per + 1) * 16, per * 16
            if extra:
                @pl.when(sid < extra)
                def _():
                    o = sid * size_a
                    pltpu.sync_copy(acc.at[pl.ds(o, size_a)],
                                    agg.at[pl.ds(base_row + o, size_a)])
            if per:
                @pl.when(sid >= extra)
                def _():
                    o = extra * size_a + (sid - extra) * size_b
                    pltpu.sync_copy(acc.at[pl.ds(o, size_b)],
                                    agg.at[pl.ds(base_row + o, size_b)])

        if split is None:
            def pass_body(p, carry):
                base_row = p * (2 * _ACC_ROWS) + cid * _ACC_ROWS
                scan_pass(base_row, _ACC_ROWS)
                write_out(base_row, _ACC_ROWS)
                plsc.subcore_barrier()
                return carry
            lax.fori_loop(0, npass, pass_body, 0)
        else:
            rows0, rows1 = split
            base_row = jnp.where(cid == 0, 0, rows0)
            rows_c = jnp.where(cid == 0, rows0, rows1)
            scan_pass(base_row, rows_c)

            @pl.when(cid == 0)
            def _():
                write_out(0, rows0)

            @pl.when(cid == 1)
            def _():
                write_out(rows0, rows1)

            plsc.subcore_barrier()

    return pl.kernel(
        body,
        out_type=jax.ShapeDtypeStruct((n_nodes, _D), jnp.float32),
        mesh=mesh,
        compiler_params=_SC_PARAMS,
        scratch_types=[
            pltpu.VMEM((ch,), jnp.int32),
            pltpu.VMEM((ch,), jnp.int32),
            pltpu.VMEM((cap,), jnp.int32),
            pltpu.VMEM((cap,), jnp.int32),
            pltpu.VMEM((cap, _D), jnp.float32),
            pltpu.VMEM_SHARED((_ACC_ALLOC, _D), jnp.float32),
            pltpu.SemaphoreType.DMA,
            pltpu.SemaphoreType.DMA,
            pltpu.SemaphoreType.DMA,
        ],
    )


def _bdot(a, b):
    return jnp.dot(a.astype(jnp.bfloat16), b.astype(jnp.bfloat16),
                   preferred_element_type=jnp.float32)


def _mlp_ln(parts, ws, b1, w2, b2, g, bt, res):
    h = b1
    for xp, w in zip(parts, ws):
        h = h + _bdot(xp, w)
    h = jnp.maximum(h, 0.0)
    h = _bdot(h, w2) + b2
    mu = jnp.mean(h, axis=-1, keepdims=True)
    var = jnp.mean((h - mu) ** 2, axis=-1, keepdims=True)
    return res + (h - mu) * lax.rsqrt(var + 1e-5) * g + bt


def _edge_mlp(gs, gd, e, wa, wb, wc, b1, w2, b2, g, bt):
    n = gs.shape[0]
    blk = 2000
    assert n % blk == 0
    rows = pl.BlockSpec((blk, _D), lambda i: (i, 0))
    wsp = pl.BlockSpec((_D, _D), lambda i: (0, 0))
    vsp = pl.BlockSpec((1, _D), lambda i: (0, 0))

    def bodyfn(gs_r, gd_r, e_r, wa_r, wb_r, wc_r, b1_r, w2_r, b2_r, g_r, bt_r,
               o_r):
        o_r[...] = _mlp_ln(
            (gs_r[...], gd_r[...], e_r[...]),
            (wa_r[...], wb_r[...], wc_r[...]),
            b1_r[...], w2_r[...], b2_r[...], g_r[...], bt_r[...], e_r[...])

    return pl.pallas_call(
        bodyfn,
        grid=(n // blk,),
        in_specs=[rows, rows, rows, wsp, wsp, wsp, vsp, wsp, vsp, vsp, vsp],
        out_specs=rows,
        out_shape=jax.ShapeDtypeStruct((n, _D), jnp.float32),
    )(gs, gd, e, wa, wb, wc, b1, w2, b2, g, bt)


def _node_mlp(x, agg, wa, wb, b1, w2, b2, g, bt):
    n = x.shape[0]
    blk = 2000
    assert n % blk == 0
    rows = pl.BlockSpec((blk, _D), lambda i: (i, 0))
    wsp = pl.BlockSpec((_D, _D), lambda i: (0, 0))
    vsp = pl.BlockSpec((1, _D), lambda i: (0, 0))

    def bodyfn(x_r, a_r, wa_r, wb_r, b1_r, w2_r, b2_r, g_r, bt_r, o_r):
        o_r[...] = _mlp_ln(
            (x_r[...], a_r[...].astype(jnp.float32)),
            (wa_r[...], wb_r[...]),
            b1_r[...], w2_r[...], b2_r[...], g_r[...], bt_r[...], x_r[...])

    return pl.pallas_call(
        bodyfn,
        grid=(n // blk,),
        in_specs=[rows, rows, wsp, wsp, vsp, wsp, vsp, vsp, vsp],
        out_specs=rows,
        out_shape=jax.ShapeDtypeStruct((n, _D), jnp.float32),
    )(x, agg, wa, wb, b1, w2, b2, g, bt)


def kernel(h_atm, h_bnd, h_ang, edge_index_bnd, edge_index_ang, ba, ab):
    zeros = jnp.zeros((_ACC_ALLOC, _D), jnp.float32)

    def conv(x, ei, e, p, i, pregather=None):
        src, dst = ei[0], ei[1]
        n_nodes, n_edges = x.shape[0], e.shape[0]
        if pregather is None:
            gs, gd = _make_gather(n_nodes, n_edges)(x, src, dst)
        else:
            gs, gd = pregather
        e2 = _edge_mlp(
            gs, gd, e,
            p['eW1'][i, :_D], p['eW1'][i, _D:2 * _D], p['eW1'][i, 2 * _D:],
            p['eb1'][i][None], p['eW2'][i], p['eb2'][i][None],
            p['eg'][i][None], p['ebt'][i][None])
        agg = _make_scatter(n_edges, n_nodes)(e2, dst, zeros)
        x2 = _node_mlp(
            x, agg,
            p['nW1'][i, :_D], p['nW1'][i, _D:],
            p['nb1'][i][None], p['nW2'][i], p['nb2'][i][None],
            p['ng'][i][None], p['nbt'][i][None])
        return x2, e2

    for i in range(2):
        pre = _make_gather(h_atm.shape[0], h_bnd.shape[0])(
            h_atm, edge_index_bnd[0], edge_index_bnd[1])
        h_bnd, h_ang = conv(h_bnd, edge_index_ang, h_ang, ba, i)
        h_atm, h_bnd = conv(h_atm, edge_index_bnd, h_bnd, ab, i,
                            pregather=pre)
    return (h_atm, h_bnd, h_ang)
